# Initial kernel scaffold; baseline (speedup 1.0000x reference)
import jax
import jax.numpy as jnp
from jax.experimental import pallas as pl


def kernel(predictions, gate_logits):
    raise NotImplementedError("write your pallas kernel here")


# speedup vs baseline: 5.0616x; 5.0616x over previous
import functools

import jax
import jax.numpy as jnp
from jax import lax
from jax.experimental import pallas as pl
from jax.experimental.pallas import tpu as pltpu
from jax.experimental.pallas import tpu_sc as plsc

_W_TOP10 = 3.0
_W_TOP11_30 = 1.5
_W_OTHERS = 0.5

_NUM_CORES = 2
_NUM_SUBCORES = 16
_LANES = 16


def _sc_row_thresholds(gate):
    n_rows, n_exp = gate.shape
    nw = _NUM_CORES * _NUM_SUBCORES
    rows_per = n_rows // nw
    mesh = plsc.VectorSubcoreMesh(core_axis_name="c", subcore_axis_name="s")

    chunk = 128

    def body(gate_hbm, t2_hbm, t10_hbm, t30_hbm, g_v, t2_v, t10_v, t30_v):
        wid = lax.axis_index("s") * _NUM_CORES + lax.axis_index("c")
        base = wid * rows_per

        def sd(x):
            return plsc.sort_key_val(x, x, descending=True)[0]

        def sa(x):
            return plsc.sort_key_val(x, x, descending=False)[0]

        lane_iota = lax.broadcasted_iota(jnp.int32, (_LANES,), 0)

        def row_stats(r):
            a = g_v[r, pl.ds(0, _LANES)]
            b = g_v[r, pl.ds(_LANES, _LANES)]
            c = g_v[r, pl.ds(2 * _LANES, _LANES)]
            d = g_v[r, pl.ds(3 * _LANES, _LANES)]
            s_ad, s_ba = sd(a), sa(b)
            s_h1 = sd(jnp.maximum(s_ad, s_ba))
            s_l1 = sd(jnp.minimum(s_ad, s_ba))
            s_ca, s_dd = sa(c), sd(d)
            t0 = sa(jnp.minimum(s_ca, s_dd))
            t1 = sa(jnp.maximum(s_ca, s_dd))
            u0 = jnp.maximum(s_h1, t0)
            u1 = jnp.maximum(s_l1, t1)
            sx = sd(jnp.maximum(u0, u1))
            sy = sd(jnp.minimum(u0, u1))
            return sx, sy

        def chunk_body(cidx, carry):
            pltpu.sync_copy(
                gate_hbm.at[pl.ds(base + cidx * chunk, chunk), :], g_v)

            def grp_body(gidx, carry2):
                base_r = gidx * _LANES
                acc2 = jnp.zeros((_LANES,), jnp.float32)
                acc10 = jnp.zeros((_LANES,), jnp.float32)
                acc30 = jnp.zeros((_LANES,), jnp.float32)
                for j in range(_LANES):
                    sx, sy = row_stats(base_r + j)
                    m = lane_iota == j
                    acc2 = jnp.where(m, sx[1], acc2)
                    acc10 = jnp.where(m, sx[9], acc10)
                    acc30 = jnp.where(m, sy[13], acc30)
                out_r = cidx * chunk + base_r
                t2_v[pl.ds(out_r, _LANES)] = acc2
                t10_v[pl.ds(out_r, _LANES)] = acc10
                t30_v[pl.ds(out_r, _LANES)] = acc30
                return carry2

            lax.fori_loop(0, chunk // _LANES, grp_body, 0)
            return carry

        lax.fori_loop(0, rows_per // chunk, chunk_body, 0)
        pltpu.sync_copy(t2_v, t2_hbm.at[pl.ds(base, rows_per)])
        pltpu.sync_copy(t10_v, t10_hbm.at[pl.ds(base, rows_per)])
        pltpu.sync_copy(t30_v, t30_hbm.at[pl.ds(base, rows_per)])

    return pl.kernel(
        body,
        out_type=[jax.ShapeDtypeStruct((n_rows,), jnp.float32)] * 3,
        mesh=mesh,
        scratch_types=[
            pltpu.VMEM((chunk, n_exp), jnp.float32),
            pltpu.VMEM((rows_per,), jnp.float32),
            pltpu.VMEM((rows_per,), jnp.float32),
            pltpu.VMEM((rows_per,), jnp.float32),
        ],
        compiler_params=pltpu.CompilerParams(needs_layout_passes=False),
    )(gate)


def _tc_loss_body(p_ref, g_ref, t2_ref, t10_ref, t30_ref, out_ref, *, inv_n):
    p = p_ref[...]
    g = g_ref[...]
    sp = jnp.maximum(p, 0.0) + jnp.log1p(jnp.exp(-jnp.abs(p)))
    coeff = (_W_OTHERS
             + (_W_TOP11_30 - _W_OTHERS)
             * (g >= t30_ref[...]).astype(jnp.float32)
             + (_W_TOP10 - _W_TOP11_30)
             * (g >= t10_ref[...]).astype(jnp.float32))
    term = coeff * sp - _W_TOP10 * jnp.where(g >= t2_ref[...], p, 0.0)
    partial = jnp.sum(term) * inv_n

    @pl.when(pl.program_id(0) == 0)
    def _():
        out_ref[...] = jnp.zeros_like(out_ref)

    out_ref[...] += partial


def _tc_loss(pred, gate, t2, t10, t30, block_rows=2048):
    n_rows, n_exp = pred.shape
    grid = (n_rows // block_rows,)
    out = pl.pallas_call(
        functools.partial(_tc_loss_body, inv_n=1.0 / (n_rows * n_exp)),
        grid=grid,
        in_specs=[
            pl.BlockSpec((block_rows, n_exp), lambda i: (i, 0)),
            pl.BlockSpec((block_rows, n_exp), lambda i: (i, 0)),
            pl.BlockSpec((block_rows, 1), lambda i: (i, 0)),
            pl.BlockSpec((block_rows, 1), lambda i: (i, 0)),
            pl.BlockSpec((block_rows, 1), lambda i: (i, 0)),
        ],
        out_specs=pl.BlockSpec((1, 1), lambda i: (0, 0)),
        out_shape=jax.ShapeDtypeStruct((1, 1), jnp.float32),
    )(pred, gate,
      t2.reshape(n_rows, 1), t10.reshape(n_rows, 1), t30.reshape(n_rows, 1))
    return out[0, 0]


def kernel(predictions, gate_logits):
    t2, t10, t30 = _sc_row_thresholds(gate_logits)
    return _tc_loss(predictions, gate_logits, t2, t10, t30)


# speedup vs baseline: 9.1494x; 1.8076x over previous
import functools

import jax
import jax.numpy as jnp
from jax import lax
from jax.experimental import pallas as pl
from jax.experimental.pallas import tpu as pltpu
from jax.experimental.pallas import tpu_sc as plsc

_NUM_CORES = 2
_NUM_SUBCORES = 16
_LANES = 16

_LOG1P_C = (
    0.010009289617861138, -0.05243753706703084, 0.13083342798333364,
    -0.22316586411879943, 0.32722571497347896, -0.49928504912250304,
    0.999967080943859, 2.5546730196161803e-07,
)


def _sc_loss(pred, gate):
    n_rows, n_exp = pred.shape
    nw = _NUM_CORES * _NUM_SUBCORES
    rows_per = n_rows // nw
    chunk = 128
    n_chunks = rows_per // chunk
    inv_n = 1.0 / float(n_rows * n_exp)
    mesh = plsc.VectorSubcoreMesh(
        core_axis_name="c", subcore_axis_name="s",
        num_cores=_NUM_CORES, num_subcores=_NUM_SUBCORES)

    def body(pred_hbm, gate_hbm, out_hbm, g_v, p_v, res_v,
             sg0, sg1, sp0, sp1):
        cid = lax.axis_index("c")
        sid = lax.axis_index("s")
        wid = sid * _NUM_CORES + cid
        base = wid * rows_per

        def g_copy(cidx, b, sem):
            return pltpu.make_async_copy(
                gate_hbm.at[pl.ds(base + cidx * chunk, chunk), :],
                g_v.at[b], sem)

        def p_copy(cidx, b, sem):
            return pltpu.make_async_copy(
                pred_hbm.at[pl.ds(base + cidx * chunk, chunk), :],
                p_v.at[b], sem)

        def sd(x):
            return plsc.sort_key_val(x, x, descending=True)[0]

        def sa(x):
            return plsc.sort_key_val(x, x, descending=False)[0]

        def row_body(b, r, acc):
            a = g_v[b, r, pl.ds(0, _LANES)]
            bb = g_v[b, r, pl.ds(_LANES, _LANES)]
            c = g_v[b, r, pl.ds(2 * _LANES, _LANES)]
            d = g_v[b, r, pl.ds(3 * _LANES, _LANES)]
            s_ad, s_ba = sd(a), sa(bb)
            s_h1 = sd(jnp.maximum(s_ad, s_ba))
            s_l1 = sd(jnp.minimum(s_ad, s_ba))
            s_ca, s_dd = sa(c), sd(d)
            t0 = sa(jnp.minimum(s_ca, s_dd))
            t1 = sa(jnp.maximum(s_ca, s_dd))
            u0 = jnp.maximum(s_h1, t0)
            u1 = jnp.maximum(s_l1, t1)
            sx = sd(jnp.maximum(u0, u1))
            sy = sd(jnp.minimum(u0, u1))
            v2 = sx[1]
            v10 = sx[9]
            v30 = sy[13]
            for q, g in enumerate((a, bb, c, d)):
                p = p_v[b, r, pl.ds(q * _LANES, _LANES)]
                coeff = (jnp.where(g >= v30, 1.5, 0.5)
                         + jnp.where(g >= v10, 1.5, 0.0))
                e = jnp.exp(-jnp.abs(p))
                poly = jnp.full((_LANES,), _LOG1P_C[0], jnp.float32)
                for cf in _LOG1P_C[1:]:
                    poly = poly * e + cf
                sp = jnp.maximum(p, 0.0) + poly
                acc = acc + coeff * sp
                acc = acc - 3.0 * jnp.where(g >= v2, p, 0.0)
            return acc

        def chunk_rows(b, acc):
            return lax.fori_loop(
                0, chunk, lambda r, ac: row_body(b, r, ac), acc)

        g_copy(0, 0, sg0).start()
        p_copy(0, 0, sp0).start()
        g_copy(1, 1, sg1).start()
        p_copy(1, 1, sp1).start()

        def outer(j, acc):
            c0 = j * 2

            g_copy(c0, 0, sg0).wait()
            p_copy(c0, 0, sp0).wait()
            acc = chunk_rows(0, acc)

            @pl.when(c0 + 2 < n_chunks)
            def _():
                g_copy(c0 + 2, 0, sg0).start()
                p_copy(c0 + 2, 0, sp0).start()

            g_copy(c0 + 1, 1, sg1).wait()
            p_copy(c0 + 1, 1, sp1).wait()
            acc = chunk_rows(1, acc)

            @pl.when(c0 + 3 < n_chunks)
            def _():
                g_copy(c0 + 3, 1, sg1).start()
                p_copy(c0 + 3, 1, sp1).start()

            return acc

        acc = jnp.zeros((_LANES,), jnp.float32)
        acc = lax.fori_loop(0, n_chunks // 2, outer, acc)

        tile_total = jnp.sum(acc) * inv_n
        res_v[...] = jnp.full((_LANES,), tile_total, jnp.float32)
        pltpu.sync_copy(res_v, out_hbm.at[wid])

    out = pl.kernel(
        body,
        out_type=jax.ShapeDtypeStruct((nw, _LANES), jnp.float32),
        mesh=mesh,
        scratch_types=[
            pltpu.VMEM((2, chunk, n_exp), jnp.float32),
            pltpu.VMEM((2, chunk, n_exp), jnp.float32),
            pltpu.VMEM((_LANES,), jnp.float32),
            pltpu.SemaphoreType.DMA,
            pltpu.SemaphoreType.DMA,
            pltpu.SemaphoreType.DMA,
            pltpu.SemaphoreType.DMA,
        ],
        compiler_params=pltpu.CompilerParams(needs_layout_passes=False),
    )(pred, gate)
    return out


def kernel(predictions, gate_logits):
    out = _sc_loss(predictions, gate_logits)
    return jnp.sum(out[:, 0])
